# Initial kernel scaffold; baseline (speedup 1.0000x reference)
import jax
import jax.numpy as jnp
from jax.experimental import pallas as pl


def kernel(loc_pred, conf_pred, refined_anchors, ignore_flags_refined_anchor, targets):
    raise NotImplementedError("write your pallas kernel here")


# speedup vs baseline: 5.9706x; 5.9706x over previous
import jax
import jax.numpy as jnp
from jax.experimental import pallas as pl
from jax.experimental.pallas import tpu as pltpu

_NUM_CLASSES = 21
_OVERLAP = 0.5
_NEG_POS = 3
_V0, _V1 = 0.1, 0.2
_S = 128
_L = 128


def _body(tgt_ref, anc_ref, loc_ref, conf_ref, ign_ref, out_ref):
    G = tgt_ref.shape[2]
    A = _S * _L
    a_lo = anc_ref[0, 0]
    a_hi = anc_ref[0, 1]
    area_a = a_hi - a_lo
    lin = (jax.lax.broadcasted_iota(jnp.int32, (_S, _L), 0) * _L
           + jax.lax.broadcasted_iota(jnp.int32, (_S, _L), 1))

    best_iou = jnp.full((_S, _L), -2.0, jnp.float32)
    best_idx = jnp.zeros((_S, _L), jnp.int32)
    g_lo, g_hi, g_lab, g_valid, g_aidx = [], [], [], [], []
    for g in range(G):
        lo = tgt_ref[0, 0, g]
        hi = tgt_ref[0, 1, g]
        lab = tgt_ref[0, 2, g]
        valid = lab > 0.0
        inter = jnp.maximum(0.0, jnp.minimum(hi, a_hi) - jnp.maximum(lo, a_lo))
        iou = inter / ((hi - lo) + area_a - inter + 1e-10)
        iou = jnp.where(valid, iou, -1.0)
        upd = iou > best_iou
        best_iou = jnp.where(upd, iou, best_iou)
        best_idx = jnp.where(upd, g, best_idx)
        m = jnp.max(iou)
        aidx = jnp.min(jnp.where(iou == m, lin, A))
        g_lo.append(lo); g_hi.append(hi); g_lab.append(lab)
        g_valid.append(valid); g_aidx.append(aidx)

    for g in range(G):
        hit = jnp.logical_and(lin == g_aidx[g], g_valid[g])
        best_iou = jnp.where(hit, 2.0, best_iou)
        best_idx = jnp.where(hit, g, best_idx)

    conf = jnp.zeros((_S, _L), jnp.float32)
    m_lo = jnp.zeros((_S, _L), jnp.float32)
    m_hi = jnp.zeros((_S, _L), jnp.float32)
    for g in range(G):
        sel = best_idx == g
        conf = jnp.where(sel, g_lab[g], conf)
        m_lo = jnp.where(sel, g_lo[g], m_lo)
        m_hi = jnp.where(sel, g_hi[g], m_hi)

    ign = ign_ref[0, 0]
    conf = jnp.where(best_iou < _OVERLAP, 0.0, conf)
    conf = jnp.where(ign > 0.0, 0.0, conf)
    pos = conf > 0.0
    posf = pos.astype(jnp.float32)
    num_pos = jnp.sum(pos.astype(jnp.int32))

    a_c = (a_lo + a_hi) * 0.5
    m_c = (m_lo + m_hi) * 0.5
    m_w = m_hi - m_lo
    enc_c = (m_c - a_c) / (_V0 * area_a)
    enc_w = jnp.log(jnp.maximum(m_w / area_a, 1e-10)) / _V1
    d0 = loc_ref[0, 0] - enc_c
    d1 = loc_ref[0, 1] - enc_w
    ad0 = jnp.abs(d0)
    ad1 = jnp.abs(d1)
    sl1 = (jnp.where(ad0 < 1.0, 0.5 * d0 * d0, ad0 - 0.5)
           + jnp.where(ad1 < 1.0, 0.5 * d1 * d1, ad1 - 0.5))
    loss_l = jnp.sum(sl1 * posf)

    mx = conf_ref[0, 0]
    for c in range(1, _NUM_CLASSES):
        mx = jnp.maximum(mx, conf_ref[0, c])
    ssum = jnp.zeros((_S, _L), jnp.float32)
    gath = jnp.zeros((_S, _L), jnp.float32)
    for c in range(_NUM_CLASSES):
        cp = conf_ref[0, c]
        ssum = ssum + jnp.exp(cp - mx)
        gath = jnp.where(conf == float(c), cp, gath)
    ce = (mx + jnp.log(ssum)) - gath

    pos_ce = jnp.sum(ce * posf)
    proxy = jnp.where(jnp.logical_or(pos, ign > 0.0), 0.0, ce)
    max_neg = jnp.sum((proxy > 0.0).astype(jnp.int32))
    k = jnp.minimum(_NEG_POS * num_pos, max_neg)

    pbits = jax.lax.bitcast_convert_type(proxy, jnp.int32)
    T = jnp.int32(0)
    for bit in range(30, -1, -1):
        cand = jnp.bitwise_or(T, jnp.int32(1 << bit))
        cnt = jnp.sum((pbits >= cand).astype(jnp.int32))
        T = jnp.where(cnt >= k, cand, T)
    tstar = jnp.max(jnp.where(pbits <= T, proxy, 0.0))
    gt = proxy > tstar
    cnt_gt = jnp.sum(gt.astype(jnp.int32))
    sum_gt = jnp.sum(jnp.where(gt, proxy, 0.0))
    topk = sum_gt + (k - cnt_gt).astype(jnp.float32) * tstar
    topk = jnp.where(k > 0, topk, 0.0)
    loss_c = pos_ce + topk

    lane = jax.lax.broadcasted_iota(jnp.int32, (1, 1, _L), 2)
    row = jnp.where(lane == 0, loss_l,
                    jnp.where(lane == 1, loss_c,
                              jnp.where(lane == 2, num_pos.astype(jnp.float32),
                                        0.0)))
    out_ref[...] = row


def kernel(loc_pred, conf_pred, refined_anchors, ignore_flags_refined_anchor, targets):
    B, A, C = conf_pred.shape
    G = targets.shape[1]
    tgt = targets.transpose(0, 2, 1)
    anc = refined_anchors.transpose(0, 2, 1).reshape(B, 2, _S, _L)
    loc = loc_pred.transpose(0, 2, 1).reshape(B, 2, _S, _L)
    cp = conf_pred.transpose(0, 2, 1).reshape(B, C, _S, _L)
    ign = ignore_flags_refined_anchor.reshape(B, 1, _S, _L)

    out = pl.pallas_call(
        _body,
        grid=(B,),
        in_specs=[
            pl.BlockSpec((1, 3, G), lambda b: (b, 0, 0)),
            pl.BlockSpec((1, 2, _S, _L), lambda b: (b, 0, 0, 0)),
            pl.BlockSpec((1, 2, _S, _L), lambda b: (b, 0, 0, 0)),
            pl.BlockSpec((1, C, _S, _L), lambda b: (b, 0, 0, 0)),
            pl.BlockSpec((1, 1, _S, _L), lambda b: (b, 0, 0, 0)),
        ],
        out_specs=pl.BlockSpec((1, 1, _L), lambda b: (b, 0, 0)),
        out_shape=jax.ShapeDtypeStruct((B, 1, _L), jnp.float32),
        compiler_params=pltpu.CompilerParams(
            dimension_semantics=("arbitrary",)),
    )(tgt, anc, loc, cp, ign)

    loss_l = jnp.sum(out[:, 0, 0])
    loss_c = jnp.sum(out[:, 0, 1])
    total = jnp.sum(out[:, 0, 2])
    return (loss_l / total, loss_c / total)


# speedup vs baseline: 12.3247x; 2.0643x over previous
import jax
import jax.numpy as jnp
from jax.experimental import pallas as pl
from jax.experimental.pallas import tpu as pltpu

_NUM_CLASSES = 21
_OVERLAP = 0.5
_NEG_POS = 3
_V0, _V1 = 0.1, 0.2
_S = 128
_L = 128


def _body(tgt_ref, anc_ref, loc_ref, conf_ref, ign_ref, out_ref):
    G = tgt_ref.shape[2]
    A = _S * _L
    a_lo = anc_ref[0, 0]
    a_hi = anc_ref[0, 1]
    area_a = a_hi - a_lo
    sub = jax.lax.broadcasted_iota(jnp.int32, (_S, _L), 0)
    lan = jax.lax.broadcasted_iota(jnp.int32, (_S, _L), 1)
    lin = sub * _L + lan

    best_iou = jnp.full((_S, _L), -2.0, jnp.float32)
    best_idx = jnp.zeros((_S, _L), jnp.int32)
    g_lo, g_hi, g_lab, g_valid = [], [], [], []
    colv_l, coli_l = [], []
    for g in range(G):
        lo = tgt_ref[0, 0, g]
        hi = tgt_ref[0, 1, g]
        lab = tgt_ref[0, 2, g]
        valid = lab > 0.0
        inter = jnp.maximum(0.0, jnp.minimum(hi, a_hi) - jnp.maximum(lo, a_lo))
        iou = inter / ((hi - lo) + area_a - inter + 1e-10)
        iou = jnp.where(valid, iou, -1.0)
        upd = iou > best_iou
        best_iou = jnp.where(upd, iou, best_iou)
        best_idx = jnp.where(upd, g, best_idx)
        colv = jnp.max(iou, axis=0, keepdims=True)
        coli = jnp.min(jnp.where(iou == colv, sub, _S),
                       axis=0, keepdims=True)
        g_lo.append(lo); g_hi.append(hi); g_lab.append(lab)
        g_valid.append(valid)
        colv_l.append(colv); coli_l.append(coli)

    M = jnp.concatenate(colv_l, axis=0)
    Sm = jnp.concatenate(coli_l, axis=0)
    lane_g = jax.lax.broadcasted_iota(jnp.int32, (G, _L), 1)
    mrow = jnp.max(M, axis=1, keepdims=True)
    linm = Sm * _L + lane_g
    aidx = jnp.min(jnp.where(M == mrow, linm, A), axis=1)

    for g in range(G):
        hit = jnp.logical_and(lin == aidx[g], g_valid[g])
        best_iou = jnp.where(hit, 2.0, best_iou)
        best_idx = jnp.where(hit, g, best_idx)

    conf = jnp.zeros((_S, _L), jnp.float32)
    m_lo = jnp.zeros((_S, _L), jnp.float32)
    m_hi = jnp.zeros((_S, _L), jnp.float32)
    for g in range(G):
        sel = best_idx == g
        conf = jnp.where(sel, g_lab[g], conf)
        m_lo = jnp.where(sel, g_lo[g], m_lo)
        m_hi = jnp.where(sel, g_hi[g], m_hi)

    ign = ign_ref[0, 0]
    conf = jnp.where(best_iou < _OVERLAP, 0.0, conf)
    conf = jnp.where(ign > 0.0, 0.0, conf)
    pos = conf > 0.0
    posf = pos.astype(jnp.float32)
    num_pos = jnp.sum(pos.astype(jnp.int32))

    a_c = (a_lo + a_hi) * 0.5
    m_c = (m_lo + m_hi) * 0.5
    m_w = m_hi - m_lo
    enc_c = (m_c - a_c) / (_V0 * area_a)
    enc_w = jnp.log(jnp.maximum(m_w / area_a, 1e-10)) / _V1
    d0 = loc_ref[0, 0] - enc_c
    d1 = loc_ref[0, 1] - enc_w
    ad0 = jnp.abs(d0)
    ad1 = jnp.abs(d1)
    sl1 = (jnp.where(ad0 < 1.0, 0.5 * d0 * d0, ad0 - 0.5)
           + jnp.where(ad1 < 1.0, 0.5 * d1 * d1, ad1 - 0.5))
    loss_l = jnp.sum(sl1 * posf)

    mx = conf_ref[0, 0]
    for c in range(1, _NUM_CLASSES):
        mx = jnp.maximum(mx, conf_ref[0, c])
    ssum = jnp.zeros((_S, _L), jnp.float32)
    gath = jnp.zeros((_S, _L), jnp.float32)
    for c in range(_NUM_CLASSES):
        cp = conf_ref[0, c]
        ssum = ssum + jnp.exp(cp - mx)
        gath = jnp.where(conf == float(c), cp, gath)
    ce = (mx + jnp.log(ssum)) - gath

    pos_ce = jnp.sum(ce * posf)
    proxy = jnp.where(jnp.logical_or(pos, ign > 0.0), 0.0, ce)
    max_neg = jnp.sum((proxy > 0.0).astype(jnp.int32))
    k = jnp.minimum(_NEG_POS * num_pos, max_neg)

    pbits = jax.lax.bitcast_convert_type(proxy, jnp.int32)
    T = jnp.int32(0)
    rounds = [(28, 7)] + [(sh, 15) for sh in range(24, -1, -4)]
    for sh, dmax in rounds:
        cnts = [jnp.sum((pbits >= jnp.bitwise_or(T, jnp.int32(d << sh)))
                        .astype(jnp.int32)) for d in range(1, dmax + 1)]
        digit = jnp.int32(0)
        for d in range(1, dmax + 1):
            digit = jnp.where(cnts[d - 1] >= k, jnp.int32(d), digit)
        T = jnp.bitwise_or(T, jnp.left_shift(digit, sh))
    tstar = jnp.max(jnp.where(pbits <= T, proxy, 0.0))
    gt = proxy > tstar
    cnt_gt = jnp.sum(gt.astype(jnp.int32))
    sum_gt = jnp.sum(jnp.where(gt, proxy, 0.0))
    topk = sum_gt + (k - cnt_gt).astype(jnp.float32) * tstar
    topk = jnp.where(k > 0, topk, 0.0)
    loss_c = pos_ce + topk

    lane3 = jax.lax.broadcasted_iota(jnp.int32, (1, 1, _L), 2)
    row = jnp.where(lane3 == 0, loss_l,
                    jnp.where(lane3 == 1, loss_c,
                              jnp.where(lane3 == 2, num_pos.astype(jnp.float32),
                                        0.0)))
    out_ref[...] = row


def kernel(loc_pred, conf_pred, refined_anchors, ignore_flags_refined_anchor, targets):
    B, A, C = conf_pred.shape
    G = targets.shape[1]
    tgt = targets.transpose(0, 2, 1)
    anc = refined_anchors.transpose(0, 2, 1).reshape(B, 2, _S, _L)
    loc = loc_pred.transpose(0, 2, 1).reshape(B, 2, _S, _L)
    cp = conf_pred.transpose(0, 2, 1).reshape(B, C, _S, _L)
    ign = ignore_flags_refined_anchor.reshape(B, 1, _S, _L)

    out = pl.pallas_call(
        _body,
        grid=(B,),
        in_specs=[
            pl.BlockSpec((1, 3, G), lambda b: (b, 0, 0),
                         memory_space=pltpu.SMEM),
            pl.BlockSpec((1, 2, _S, _L), lambda b: (b, 0, 0, 0)),
            pl.BlockSpec((1, 2, _S, _L), lambda b: (b, 0, 0, 0)),
            pl.BlockSpec((1, C, _S, _L), lambda b: (b, 0, 0, 0)),
            pl.BlockSpec((1, 1, _S, _L), lambda b: (b, 0, 0, 0)),
        ],
        out_specs=pl.BlockSpec((1, 1, _L), lambda b: (b, 0, 0)),
        out_shape=jax.ShapeDtypeStruct((B, 1, _L), jnp.float32),
        compiler_params=pltpu.CompilerParams(
            dimension_semantics=("arbitrary",)),
    )(tgt, anc, loc, cp, ign)

    loss_l = jnp.sum(out[:, 0, 0])
    loss_c = jnp.sum(out[:, 0, 1])
    total = jnp.sum(out[:, 0, 2])
    return (loss_l / total, loss_c / total)


# speedup vs baseline: 14.7496x; 1.1967x over previous
import jax
import jax.numpy as jnp
from jax.experimental import pallas as pl
from jax.experimental.pallas import tpu as pltpu

_NUM_CLASSES = 21
_OVERLAP = 0.5
_NEG_POS = 3
_V0, _V1 = 0.1, 0.2
_S = 128
_L = 128
_B = 8
_RADIX_ROUNDS = [(28, 7)] + [(sh, 15) for sh in range(24, -1, -4)]


def _phase1(b, tgt_ref, anc_ref, loc_ref, conf_ref, ign_ref,
            proxy_s, pll_s, pce_s, pnp_s, pmn_s):
    G = tgt_ref.shape[2]
    A = _S * _L
    a_lo = anc_ref[0, 0]
    a_hi = anc_ref[0, 1]
    area_a = a_hi - a_lo
    sub = jax.lax.broadcasted_iota(jnp.int32, (_S, _L), 0)
    lan = jax.lax.broadcasted_iota(jnp.int32, (_S, _L), 1)
    lin = sub * _L + lan

    best_iou = jnp.full((_S, _L), -2.0, jnp.float32)
    best_idx = jnp.zeros((_S, _L), jnp.int32)
    g_lo, g_hi, g_lab, g_valid = [], [], [], []
    colv_l, coli_l = [], []
    for g in range(G):
        lo = tgt_ref[0, 0, g]
        hi = tgt_ref[0, 1, g]
        lab = tgt_ref[0, 2, g]
        valid = lab > 0.0
        inter = jnp.maximum(0.0, jnp.minimum(hi, a_hi) - jnp.maximum(lo, a_lo))
        iou = inter / ((hi - lo) + area_a - inter + 1e-10)
        iou = jnp.where(valid, iou, -1.0)
        upd = iou > best_iou
        best_iou = jnp.where(upd, iou, best_iou)
        best_idx = jnp.where(upd, g, best_idx)
        colv = jnp.max(iou, axis=0, keepdims=True)
        coli = jnp.min(jnp.where(iou == colv, sub, _S),
                       axis=0, keepdims=True)
        g_lo.append(lo); g_hi.append(hi); g_lab.append(lab)
        g_valid.append(valid)
        colv_l.append(colv); coli_l.append(coli)

    M = jnp.concatenate(colv_l, axis=0)
    Sm = jnp.concatenate(coli_l, axis=0)
    lane_g = jax.lax.broadcasted_iota(jnp.int32, (G, _L), 1)
    mrow = jnp.max(M, axis=1, keepdims=True)
    linm = Sm * _L + lane_g
    aidx = jnp.min(jnp.where(M == mrow, linm, A), axis=1)

    for g in range(G):
        hit = jnp.logical_and(lin == aidx[g], g_valid[g])
        best_iou = jnp.where(hit, 2.0, best_iou)
        best_idx = jnp.where(hit, g, best_idx)

    conf = jnp.zeros((_S, _L), jnp.float32)
    m_lo = jnp.zeros((_S, _L), jnp.float32)
    m_hi = jnp.zeros((_S, _L), jnp.float32)
    for g in range(G):
        sel = best_idx == g
        conf = jnp.where(sel, g_lab[g], conf)
        m_lo = jnp.where(sel, g_lo[g], m_lo)
        m_hi = jnp.where(sel, g_hi[g], m_hi)

    ign = ign_ref[0, 0]
    conf = jnp.where(best_iou < _OVERLAP, 0.0, conf)
    conf = jnp.where(ign > 0.0, 0.0, conf)
    pos = conf > 0.0
    posf = pos.astype(jnp.float32)

    a_c = (a_lo + a_hi) * 0.5
    m_c = (m_lo + m_hi) * 0.5
    m_w = m_hi - m_lo
    enc_c = (m_c - a_c) / (_V0 * area_a)
    enc_w = jnp.log(jnp.maximum(m_w / area_a, 1e-10)) / _V1
    d0 = loc_ref[0, 0] - enc_c
    d1 = loc_ref[0, 1] - enc_w
    ad0 = jnp.abs(d0)
    ad1 = jnp.abs(d1)
    sl1 = (jnp.where(ad0 < 1.0, 0.5 * d0 * d0, ad0 - 0.5)
           + jnp.where(ad1 < 1.0, 0.5 * d1 * d1, ad1 - 0.5))

    gmx = jnp.max(conf_ref[0, 0])
    for c in range(1, _NUM_CLASSES):
        gmx = jnp.maximum(gmx, jnp.max(conf_ref[0, c]))
    ssum = jnp.zeros((_S, _L), jnp.float32)
    gath = jnp.zeros((_S, _L), jnp.float32)
    for c in range(_NUM_CLASSES):
        cp = conf_ref[0, c]
        ssum = ssum + jnp.exp(cp - gmx)
        gath = jnp.where(conf == float(c), cp, gath)
    ce = (gmx + jnp.log(ssum)) - gath

    proxy = jnp.where(jnp.logical_or(pos, ign > 0.0), 0.0, ce)

    proxy_s[b] = proxy
    pll_s[b, :] = jnp.sum(sl1 * posf, axis=0)
    pce_s[b, :] = jnp.sum(ce * posf, axis=0)
    pnp_s[b, :] = jnp.sum(posf, axis=0)
    pmn_s[b, :] = jnp.sum(jnp.where(proxy > 0.0, 1.0, 0.0), axis=0)


def _phase2(out_ref, proxy_s, pll_s, pce_s, pnp_s, pmn_s):
    np_r = jnp.sum(pnp_s[...], axis=1)
    mn_r = jnp.sum(pmn_s[...], axis=1)
    k_r = jnp.minimum(float(_NEG_POS) * np_r, mn_r)
    ll_sum = jnp.sum(pll_s[...])
    ce_pos_sum = jnp.sum(pce_s[...])
    total = jnp.sum(np_r)

    ks = [k_r[r] for r in range(_B)]
    Ts = [jnp.int32(0) for _ in range(_B)]
    for sh, dmax in _RADIX_ROUNDS:
        for r in range(_B):
            pb = jax.lax.bitcast_convert_type(proxy_s[r], jnp.int32)
            cnts = [jnp.sum(jnp.where(
                pb >= jnp.bitwise_or(Ts[r], jnp.int32(d << sh)), 1.0, 0.0))
                for d in range(1, dmax + 1)]
            digit = jnp.int32(0)
            for d in range(1, dmax + 1):
                digit = jnp.where(cnts[d - 1] >= ks[r], jnp.int32(d), digit)
            Ts[r] = jnp.bitwise_or(Ts[r], jnp.left_shift(digit, sh))

    topk_total = jnp.float32(0.0)
    for r in range(_B):
        proxy = proxy_s[r]
        pb = jax.lax.bitcast_convert_type(proxy, jnp.int32)
        tstar = jnp.max(jnp.where(pb <= Ts[r], proxy, 0.0))
        gtm = proxy > tstar
        cnt_gt = jnp.sum(jnp.where(gtm, 1.0, 0.0))
        sum_gt = jnp.sum(jnp.where(gtm, proxy, 0.0))
        topk = sum_gt + (ks[r] - cnt_gt) * tstar
        topk_total = topk_total + jnp.where(ks[r] > 0, topk, 0.0)

    loss_l = ll_sum / total
    loss_c = (ce_pos_sum + topk_total) / total
    lane3 = jax.lax.broadcasted_iota(jnp.int32, (1, 1, _L), 2)
    out_ref[...] = jnp.where(lane3 == 0, loss_l,
                             jnp.where(lane3 == 1, loss_c, 0.0))


def _body(tgt_ref, anc_ref, loc_ref, conf_ref, ign_ref, out_ref,
          proxy_s, pll_s, pce_s, pnp_s, pmn_s):
    b = pl.program_id(0)

    @pl.when(b < _B)
    def _():
        _phase1(b, tgt_ref, anc_ref, loc_ref, conf_ref, ign_ref,
                proxy_s, pll_s, pce_s, pnp_s, pmn_s)

    @pl.when(b == _B)
    def _():
        _phase2(out_ref, proxy_s, pll_s, pce_s, pnp_s, pmn_s)


def kernel(loc_pred, conf_pred, refined_anchors, ignore_flags_refined_anchor, targets):
    B, A, C = conf_pred.shape
    G = targets.shape[1]
    tgt = targets.transpose(0, 2, 1)
    anc = refined_anchors.transpose(0, 2, 1).reshape(B, 2, _S, _L)
    loc = loc_pred.transpose(0, 2, 1).reshape(B, 2, _S, _L)
    cp = conf_pred.transpose(0, 2, 1).reshape(B, C, _S, _L)
    ign = ignore_flags_refined_anchor.reshape(B, 1, _S, _L)

    def idx(b):
        c = jnp.minimum(b, _B - 1)
        return (c, 0, 0, 0)

    out = pl.pallas_call(
        _body,
        grid=(B + 1,),
        in_specs=[
            pl.BlockSpec((1, 3, G), lambda b: (jnp.minimum(b, _B - 1), 0, 0),
                         memory_space=pltpu.SMEM),
            pl.BlockSpec((1, 2, _S, _L), idx),
            pl.BlockSpec((1, 2, _S, _L), idx),
            pl.BlockSpec((1, C, _S, _L), idx),
            pl.BlockSpec((1, 1, _S, _L), idx),
        ],
        out_specs=pl.BlockSpec((1, 1, _L), lambda b: (0, 0, 0)),
        out_shape=jax.ShapeDtypeStruct((1, 1, _L), jnp.float32),
        scratch_shapes=[
            pltpu.VMEM((_B, _S, _L), jnp.float32),
            pltpu.VMEM((_B, _L), jnp.float32),
            pltpu.VMEM((_B, _L), jnp.float32),
            pltpu.VMEM((_B, _L), jnp.float32),
            pltpu.VMEM((_B, _L), jnp.float32),
        ],
        compiler_params=pltpu.CompilerParams(
            dimension_semantics=("arbitrary",)),
    )(tgt, anc, loc, cp, ign)

    return (out[0, 0, 0], out[0, 0, 1])


# speedup vs baseline: 15.7437x; 1.0674x over previous
import jax
import jax.numpy as jnp
from jax.experimental import pallas as pl
from jax.experimental.pallas import tpu as pltpu

_NUM_CLASSES = 21
_OVERLAP = 0.5
_NEG_POS = 3
_V0, _V1 = 0.1, 0.2
_S = 128
_L = 128
_B = 8


def _phase1(b, tgt_ref, anc_ref, loc_ref, conf_ref, ign_ref,
            proxy_s, pll_s, pce_s, pnp_s, pmn_s):
    G = tgt_ref.shape[2]
    A = _S * _L
    a_lo = anc_ref[0, 0]
    a_hi = anc_ref[0, 1]
    area_a = a_hi - a_lo
    sub = jax.lax.broadcasted_iota(jnp.int32, (_S, _L), 0)
    lan = jax.lax.broadcasted_iota(jnp.int32, (_S, _L), 1)
    lin = sub * _L + lan

    best_iou = jnp.full((_S, _L), -2.0, jnp.float32)
    best_idx = jnp.zeros((_S, _L), jnp.int32)
    g_lo, g_hi, g_lab, g_valid = [], [], [], []
    colv_l, coli_l = [], []
    for g in range(G):
        lo = tgt_ref[0, 0, g]
        hi = tgt_ref[0, 1, g]
        lab = tgt_ref[0, 2, g]
        valid = lab > 0.0
        inter = jnp.maximum(0.0, jnp.minimum(hi, a_hi) - jnp.maximum(lo, a_lo))
        iou = inter / ((hi - lo) + area_a - inter + 1e-10)
        iou = jnp.where(valid, iou, -1.0)
        upd = iou > best_iou
        best_iou = jnp.where(upd, iou, best_iou)
        best_idx = jnp.where(upd, g, best_idx)
        colv = jnp.max(iou, axis=0, keepdims=True)
        coli = jnp.min(jnp.where(iou == colv, sub, _S),
                       axis=0, keepdims=True)
        g_lo.append(lo); g_hi.append(hi); g_lab.append(lab)
        g_valid.append(valid)
        colv_l.append(colv); coli_l.append(coli)

    M = jnp.concatenate(colv_l, axis=0)
    Sm = jnp.concatenate(coli_l, axis=0)
    lane_g = jax.lax.broadcasted_iota(jnp.int32, (G, _L), 1)
    mrow = jnp.max(M, axis=1, keepdims=True)
    linm = Sm * _L + lane_g
    aidx = jnp.min(jnp.where(M == mrow, linm, A), axis=1)

    for g in range(G):
        hit = lin == jnp.where(g_valid[g], aidx[g], -1)
        best_iou = jnp.where(hit, 2.0, best_iou)
        best_idx = jnp.where(hit, g, best_idx)

    conf = jnp.zeros((_S, _L), jnp.float32)
    m_lo = jnp.zeros((_S, _L), jnp.float32)
    m_hi = jnp.zeros((_S, _L), jnp.float32)
    for g in range(G):
        sel = best_idx == g
        conf = jnp.where(sel, g_lab[g], conf)
        m_lo = jnp.where(sel, g_lo[g], m_lo)
        m_hi = jnp.where(sel, g_hi[g], m_hi)

    ign = ign_ref[0, 0]
    conf = jnp.where(best_iou < _OVERLAP, 0.0, conf)
    conf = jnp.where(ign > 0.0, 0.0, conf)
    pos = conf > 0.0
    posf = pos.astype(jnp.float32)

    a_c = (a_lo + a_hi) * 0.5
    m_c = (m_lo + m_hi) * 0.5
    m_w = m_hi - m_lo
    enc_c = (m_c - a_c) / (_V0 * area_a)
    enc_w = jnp.log(jnp.maximum(m_w / area_a, 1e-10)) / _V1
    d0 = loc_ref[0, 0] - enc_c
    d1 = loc_ref[0, 1] - enc_w
    ad0 = jnp.abs(d0)
    ad1 = jnp.abs(d1)
    sl1 = (jnp.where(ad0 < 1.0, 0.5 * d0 * d0, ad0 - 0.5)
           + jnp.where(ad1 < 1.0, 0.5 * d1 * d1, ad1 - 0.5))

    vm = conf_ref[0, 0]
    for c in range(1, _NUM_CLASSES):
        vm = jnp.maximum(vm, conf_ref[0, c])
    gmx = jnp.max(vm)
    ssum = jnp.zeros((_S, _L), jnp.float32)
    gath = jnp.zeros((_S, _L), jnp.float32)
    for c in range(_NUM_CLASSES):
        cp = conf_ref[0, c]
        ssum = ssum + jnp.exp(cp - gmx)
        gath = jnp.where(conf == float(c), cp, gath)
    ce = (gmx + jnp.log(ssum)) - gath

    proxy = jnp.where(jnp.logical_or(pos, ign > 0.0), 0.0, ce)

    proxy_s[b] = proxy
    pll_s[b, :] = jnp.sum(sl1 * posf, axis=0)
    pce_s[b, :] = jnp.sum(ce * posf, axis=0)
    pnp_s[b, :] = jnp.sum(posf, axis=0)
    pmn_s[b, :] = jnp.sum(jnp.where(proxy > 0.0, 1.0, 0.0), axis=0)


def _phase2(out_ref, proxy_s, pll_s, pce_s, pnp_s, pmn_s):
    np_r = jnp.sum(pnp_s[...], axis=1)
    mn_r = jnp.sum(pmn_s[...], axis=1)
    k_r = jnp.minimum(float(_NEG_POS) * np_r, mn_r)
    ll_sum = jnp.sum(pll_s[...])
    ce_pos_sum = jnp.sum(pce_s[...])
    total = jnp.sum(np_r)

    ks = [k_r[r] for r in range(_B)]
    pbs = [jax.lax.bitcast_convert_type(proxy_s[r], jnp.int32)
           for r in range(_B)]
    Ts = [jnp.int32(0) for _ in range(_B)]
    for bit in range(30, -1, -1):
        for r in range(_B):
            cand = jnp.bitwise_or(Ts[r], jnp.int32(1 << bit))
            cnt = jnp.sum(jnp.where(pbs[r] >= cand, 1.0, 0.0))
            Ts[r] = jnp.where(cnt >= ks[r], cand, Ts[r])

    topk_total = jnp.float32(0.0)
    for r in range(_B):
        proxy = proxy_s[r]
        pb = jax.lax.bitcast_convert_type(proxy, jnp.int32)
        tstar = jnp.max(jnp.where(pb <= Ts[r], proxy, 0.0))
        gtm = proxy > tstar
        cnt_gt = jnp.sum(jnp.where(gtm, 1.0, 0.0))
        sum_gt = jnp.sum(jnp.where(gtm, proxy, 0.0))
        topk = sum_gt + (ks[r] - cnt_gt) * tstar
        topk_total = topk_total + jnp.where(ks[r] > 0, topk, 0.0)

    loss_l = ll_sum / total
    loss_c = (ce_pos_sum + topk_total) / total
    lane3 = jax.lax.broadcasted_iota(jnp.int32, (1, 1, _L), 2)
    out_ref[...] = jnp.where(lane3 == 0, loss_l,
                             jnp.where(lane3 == 1, loss_c, 0.0))


def _body(tgt_ref, anc_ref, loc_ref, conf_ref, ign_ref, out_ref,
          proxy_s, pll_s, pce_s, pnp_s, pmn_s):
    b = pl.program_id(0)

    @pl.when(b < _B)
    def _():
        _phase1(b, tgt_ref, anc_ref, loc_ref, conf_ref, ign_ref,
                proxy_s, pll_s, pce_s, pnp_s, pmn_s)

    @pl.when(b == _B)
    def _():
        _phase2(out_ref, proxy_s, pll_s, pce_s, pnp_s, pmn_s)


def kernel(loc_pred, conf_pred, refined_anchors, ignore_flags_refined_anchor, targets):
    B, A, C = conf_pred.shape
    G = targets.shape[1]
    tgt = targets.transpose(0, 2, 1)
    anc = refined_anchors.transpose(0, 2, 1).reshape(B, 2, _S, _L)
    loc = loc_pred.transpose(0, 2, 1).reshape(B, 2, _S, _L)
    cp = conf_pred.transpose(0, 2, 1).reshape(B, C, _S, _L)
    ign = ignore_flags_refined_anchor.reshape(B, 1, _S, _L)

    def idx(b):
        c = jnp.minimum(b, _B - 1)
        return (c, 0, 0, 0)

    out = pl.pallas_call(
        _body,
        grid=(B + 1,),
        in_specs=[
            pl.BlockSpec((1, 3, G), lambda b: (jnp.minimum(b, _B - 1), 0, 0),
                         memory_space=pltpu.SMEM),
            pl.BlockSpec((1, 2, _S, _L), idx),
            pl.BlockSpec((1, 2, _S, _L), idx),
            pl.BlockSpec((1, C, _S, _L), idx),
            pl.BlockSpec((1, 1, _S, _L), idx),
        ],
        out_specs=pl.BlockSpec((1, 1, _L), lambda b: (0, 0, 0)),
        out_shape=jax.ShapeDtypeStruct((1, 1, _L), jnp.float32),
        scratch_shapes=[
            pltpu.VMEM((_B, _S, _L), jnp.float32),
            pltpu.VMEM((_B, _L), jnp.float32),
            pltpu.VMEM((_B, _L), jnp.float32),
            pltpu.VMEM((_B, _L), jnp.float32),
            pltpu.VMEM((_B, _L), jnp.float32),
        ],
        compiler_params=pltpu.CompilerParams(
            dimension_semantics=("arbitrary",)),
    )(tgt, anc, loc, cp, ign)

    return (out[0, 0, 0], out[0, 0, 1])
